# baseline (device time: 345767 ns/iter reference)
import jax
import jax.numpy as jnp
from jax import lax
from jax.experimental import pallas as pl
from jax.experimental.pallas import tpu as pltpu

B = 4
S = 1024
H_SHARD = 16
D = 128
K = H_SHARD * D
N = 4096
S_HALF = S // 2
NT = 2048


def kernel(O, Wo):
    o = O.reshape(B, S, K).astype(jnp.bfloat16)
    w = Wo.astype(jnp.bfloat16)

    def body(
        o_hbm, w_ref, out_hbm,
        o_tile, send_buf, recv_buf, out_stage,
        load_sem, store_sem, send_sems, recv_sems, credit_sem,
    ):
        my_x = lax.axis_index("x")
        my_y = lax.axis_index("y")
        my_z = lax.axis_index("z")
        partner = (my_x, my_y, 1 - my_z)

        barrier_sem = pltpu.get_barrier_semaphore()
        pl.semaphore_signal(
            barrier_sem, inc=1,
            device_id=partner, device_id_type=pl.DeviceIdType.MESH,
        )
        pl.semaphore_wait(barrier_sem, 1)

        send_off = (1 - my_z) * S_HALF
        keep_off = my_z * S_HALF

        rdmas = []
        for b in range(B):
            slot = b % 2

            cp = pltpu.make_async_copy(
                o_hbm.at[b, pl.ds(send_off, S_HALF), :],
                o_tile, load_sem,
            )
            cp.start()
            cp.wait()
            if b >= 2:
                pl.semaphore_wait(credit_sem, 1)
                rdmas[b - 2].wait_send()
            for n in range(N // NT):
                send_buf[slot, :, n * NT:(n + 1) * NT] = jnp.dot(
                    o_tile[...], w_ref[:, n * NT:(n + 1) * NT],
                    preferred_element_type=jnp.float32,
                ).astype(jnp.bfloat16)
            rdma = pltpu.make_async_remote_copy(
                src_ref=send_buf.at[slot],
                dst_ref=recv_buf.at[slot],
                send_sem=send_sems.at[slot],
                recv_sem=recv_sems.at[slot],
                device_id=partner,
                device_id_type=pl.DeviceIdType.MESH,
            )
            rdma.start()
            rdmas.append(rdma)

            cp = pltpu.make_async_copy(
                o_hbm.at[b, pl.ds(keep_off, S_HALF), :],
                o_tile, load_sem,
            )
            cp.start()
            cp.wait()
            for n in range(N // NT):
                out_stage[...] = jnp.dot(
                    o_tile[...], w_ref[:, n * NT:(n + 1) * NT],
                    preferred_element_type=jnp.float32,
                )
                if n == 0:
                    rdma.wait_recv()
                out_stage[...] = out_stage[...] + recv_buf[
                    slot, :, n * NT:(n + 1) * NT
                ].astype(jnp.float32)
                st = pltpu.make_async_copy(
                    out_stage,
                    out_hbm.at[b, :, pl.ds(n * NT, NT)],
                    store_sem,
                )
                st.start()
                st.wait()
            pl.semaphore_signal(
                credit_sem, inc=1,
                device_id=partner, device_id_type=pl.DeviceIdType.MESH,
            )

        rdmas[B - 2].wait_send()
        rdmas[B - 1].wait_send()
        pl.semaphore_wait(credit_sem, 2)

    return pl.pallas_call(
        body,
        out_shape=jax.ShapeDtypeStruct((B, S_HALF, N), jnp.float32),
        in_specs=[
            pl.BlockSpec(memory_space=pl.ANY),
            pl.BlockSpec(memory_space=pltpu.VMEM),
        ],
        out_specs=pl.BlockSpec(memory_space=pl.ANY),
        scratch_shapes=[
            pltpu.VMEM((S_HALF, K), jnp.bfloat16),
            pltpu.VMEM((2, S_HALF, N), jnp.bfloat16),
            pltpu.VMEM((2, S_HALF, N), jnp.bfloat16),
            pltpu.VMEM((S_HALF, NT), jnp.float32),
            pltpu.SemaphoreType.DMA,
            pltpu.SemaphoreType.DMA,
            pltpu.SemaphoreType.DMA((2,)),
            pltpu.SemaphoreType.DMA((2,)),
            pltpu.SemaphoreType.REGULAR,
        ],
        compiler_params=pltpu.CompilerParams(
            collective_id=0,
            vmem_limit_bytes=47 * 1024 * 1024,
        ),
    )(o, w)


# device time: 275720 ns/iter; 1.2541x vs baseline; 1.2541x over previous
import jax
import jax.numpy as jnp
from jax import lax
from jax.experimental import pallas as pl
from jax.experimental.pallas import tpu as pltpu

B = 4
S = 1024
H_SHARD = 16
D = 128
K = H_SHARD * D
N = 4096
S_HALF = S // 2
NT = 2048


def kernel(O, Wo):
    o = O.reshape(B, S, K).astype(jnp.bfloat16)
    w = Wo.astype(jnp.bfloat16)

    def body(
        o_hbm, w_ref, out_hbm,
        o_send, o_keep, send_buf, recv_buf, out_stage,
        load_sem, keep_sem, store_sem, send_sems, recv_sems, credit_sem,
    ):
        my_x = lax.axis_index("x")
        my_y = lax.axis_index("y")
        my_z = lax.axis_index("z")
        partner = (my_x, my_y, 1 - my_z)

        barrier_sem = pltpu.get_barrier_semaphore()
        pl.semaphore_signal(
            barrier_sem, inc=1,
            device_id=partner, device_id_type=pl.DeviceIdType.MESH,
        )
        pl.semaphore_wait(barrier_sem, 1)

        send_off = (1 - my_z) * S_HALF
        keep_off = my_z * S_HALF

        def consume(b, rdma_b):
            slot = b % 2
            for n in range(N // NT):
                out_stage[...] = jnp.dot(
                    o_keep[...], w_ref[:, n * NT:(n + 1) * NT],
                    preferred_element_type=jnp.float32,
                )
                if n == 0:
                    rdma_b.wait_recv()
                out_stage[...] = out_stage[...] + recv_buf[
                    slot, :, n * NT:(n + 1) * NT
                ].astype(jnp.float32)
                st = pltpu.make_async_copy(
                    out_stage,
                    out_hbm.at[b, :, pl.ds(n * NT, NT)],
                    store_sem,
                )
                st.start()
                st.wait()
            pl.semaphore_signal(
                credit_sem, inc=1,
                device_id=partner, device_id_type=pl.DeviceIdType.MESH,
            )

        rdmas = []
        for b in range(B):
            slot = b % 2

            cp_s = pltpu.make_async_copy(
                o_hbm.at[b, pl.ds(send_off, S_HALF), :],
                o_send, load_sem,
            )
            cp_s.start()
            if b >= 1:
                cp_k = pltpu.make_async_copy(
                    o_hbm.at[b - 1, pl.ds(keep_off, S_HALF), :],
                    o_keep, keep_sem,
                )
                cp_k.start()
            cp_s.wait()
            if b >= 2:
                pl.semaphore_wait(credit_sem, 1)
                rdmas[b - 2].wait_send()
            for n in range(N // NT):
                send_buf[slot, :, n * NT:(n + 1) * NT] = jnp.dot(
                    o_send[...], w_ref[:, n * NT:(n + 1) * NT],
                    preferred_element_type=jnp.float32,
                ).astype(jnp.bfloat16)
            rdma = pltpu.make_async_remote_copy(
                src_ref=send_buf.at[slot],
                dst_ref=recv_buf.at[slot],
                send_sem=send_sems.at[slot],
                recv_sem=recv_sems.at[slot],
                device_id=partner,
                device_id_type=pl.DeviceIdType.MESH,
            )
            rdma.start()
            rdmas.append(rdma)

            if b >= 1:
                cp_k.wait()
                consume(b - 1, rdmas[b - 1])

        cp_k = pltpu.make_async_copy(
            o_hbm.at[B - 1, pl.ds(keep_off, S_HALF), :],
            o_keep, keep_sem,
        )
        cp_k.start()
        cp_k.wait()
        consume(B - 1, rdmas[B - 1])

        rdmas[B - 2].wait_send()
        rdmas[B - 1].wait_send()
        pl.semaphore_wait(credit_sem, 2)

    return pl.pallas_call(
        body,
        out_shape=jax.ShapeDtypeStruct((B, S_HALF, N), jnp.float32),
        in_specs=[
            pl.BlockSpec(memory_space=pl.ANY),
            pl.BlockSpec(memory_space=pltpu.VMEM),
        ],
        out_specs=pl.BlockSpec(memory_space=pl.ANY),
        scratch_shapes=[
            pltpu.VMEM((S_HALF, K), jnp.bfloat16),
            pltpu.VMEM((S_HALF, K), jnp.bfloat16),
            pltpu.VMEM((2, S_HALF, N), jnp.bfloat16),
            pltpu.VMEM((2, S_HALF, N), jnp.bfloat16),
            pltpu.VMEM((S_HALF, NT), jnp.float32),
            pltpu.SemaphoreType.DMA,
            pltpu.SemaphoreType.DMA,
            pltpu.SemaphoreType.DMA,
            pltpu.SemaphoreType.DMA((2,)),
            pltpu.SemaphoreType.DMA((2,)),
            pltpu.SemaphoreType.REGULAR,
        ],
        compiler_params=pltpu.CompilerParams(
            collective_id=0,
            vmem_limit_bytes=47 * 1024 * 1024,
        ),
    )(o, w)
